# baseline (speedup 1.0000x reference)
import jax
import jax.numpy as jnp
from jax.experimental import pallas as pl
from jax.experimental.pallas import tpu as pltpu


def _router_block(x_ref, w_ref, b_ref, logits_ref, probs_ref):
    logits = jnp.dot(x_ref[...], w_ref[...], preferred_element_type=jnp.float32)
    logits = logits + b_ref[...]
    logits_ref[...] = logits
    m = jnp.max(logits, axis=-1, keepdims=True)
    e = jnp.exp(logits - m)
    probs_ref[...] = e / jnp.sum(e, axis=-1, keepdims=True)


def kernel(x, W, b):
    tokens, d = x.shape
    n_adapters = W.shape[1]
    bt = 4096
    b2 = b.reshape(1, n_adapters)
    out_shape = jax.ShapeDtypeStruct((tokens, n_adapters), jnp.float32)
    logits, probs = pl.pallas_call(
        _router_block,
        grid=(tokens // bt,),
        in_specs=[
            pl.BlockSpec((bt, d), lambda i: (i, 0)),
            pl.BlockSpec((d, n_adapters), lambda i: (0, 0)),
            pl.BlockSpec((1, n_adapters), lambda i: (0, 0)),
        ],
        out_specs=[
            pl.BlockSpec((bt, n_adapters), lambda i: (i, 0)),
            pl.BlockSpec((bt, n_adapters), lambda i: (i, 0)),
        ],
        out_shape=[out_shape, out_shape],
        compiler_params=pltpu.CompilerParams(
            dimension_semantics=(pltpu.PARALLEL,),
            vmem_limit_bytes=100 * 1024 * 1024,
        ),
    )(x, W, b2)
    return (logits, probs)


# speedup vs baseline: 1.0042x; 1.0042x over previous
import jax
import jax.numpy as jnp
from jax.experimental import pallas as pl
from jax.experimental.pallas import tpu as pltpu

NSPLIT = 2
BT = 4096


def _router_block(*refs):
    x_refs = refs[:NSPLIT]
    w_ref, b_ref = refs[NSPLIT], refs[NSPLIT + 1]
    logits_ref, probs_ref = refs[NSPLIT + 2], refs[NSPLIT + 3]
    sub = BT // NSPLIT
    w = w_ref[...]
    bias = b_ref[...]
    for s in range(NSPLIT):
        logits = jnp.dot(x_refs[s][...], w, preferred_element_type=jnp.float32)
        logits = logits + bias
        logits_ref[pl.ds(s * sub, sub), :] = logits
        m = jnp.max(logits, axis=-1, keepdims=True)
        e = jnp.exp(logits - m)
        probs_ref[pl.ds(s * sub, sub), :] = e / jnp.sum(e, axis=-1, keepdims=True)


def kernel(x, W, b):
    tokens, d = x.shape
    n_adapters = W.shape[1]
    b2 = b.reshape(1, n_adapters)
    out_shape = jax.ShapeDtypeStruct((tokens, n_adapters), jnp.float32)
    sub = BT // NSPLIT

    def x_spec(s):
        return pl.BlockSpec((sub, d), lambda i, s=s: (NSPLIT * i + s, 0))

    logits, probs = pl.pallas_call(
        _router_block,
        grid=(tokens // BT,),
        in_specs=[x_spec(s) for s in range(NSPLIT)]
        + [
            pl.BlockSpec((d, n_adapters), lambda i: (0, 0)),
            pl.BlockSpec((1, n_adapters), lambda i: (0, 0)),
        ],
        out_specs=[
            pl.BlockSpec((BT, n_adapters), lambda i: (i, 0)),
            pl.BlockSpec((BT, n_adapters), lambda i: (i, 0)),
        ],
        out_shape=[out_shape, out_shape],
        compiler_params=pltpu.CompilerParams(
            dimension_semantics=(pltpu.PARALLEL,),
            vmem_limit_bytes=100 * 1024 * 1024,
        ),
    )(*([x] * NSPLIT), W, b2)
    return (logits, probs)
